# baseline (speedup 1.0000x reference)
import functools

import jax
import jax.numpy as jnp
from jax import lax
from jax.experimental import pallas as pl
from jax.experimental.pallas import tpu as pltpu
from jax.experimental.pallas import tpu_sc as plsc

_MASK_RATIO = 0.25
_NLANE = 16
_ROUNDS = ((19, 4), (15, 4), (11, 4), (7, 4), (3, 4), (0, 3))


def _sc_body(k, rows_per_worker, nvec, x_hbm, o_hbm, xrow, orow, h8, s8,
             ca, pa, cb, pb, h4, s4):
    c = lax.axis_index("c")
    s = lax.axis_index("s")
    wid = s * 2 + c
    lane = lax.iota(jnp.int32, _NLANE)
    lane256 = lane * 256
    lane16 = lane * _NLANE
    ones_i = jnp.ones((_NLANE,), jnp.int32)
    zeros_i = jnp.zeros((_NLANE,), jnp.int32)
    ones_f = jnp.ones((_NLANE,), jnp.float32)

    def _abits(i):
        v = xrow[pl.ds(i * _NLANE, _NLANE)]
        return lax.bitcast_convert_type(v, jnp.int32) & 0x7FFFFFFF

    def _zero_h8(i, carry):
        h8[pl.ds(i * _NLANE, _NLANE)] = zeros_i
        return carry

    lax.fori_loop(0, 256, _zero_h8, 0)
    s8[pl.ds(256, _NLANE)] = zeros_i
    s4[pl.ds(_NLANE, _NLANE)] = zeros_i

    def row_body(r, carry):
        row = wid * rows_per_worker + r
        pltpu.sync_copy(x_hbm.at[row], xrow)

        def _hist8(i, carry):
            for u in range(8):
                a = _abits(i * 8 + u)
                plsc.addupdate_scatter(h8, [lane256 + (a >> 23)], ones_i)
            return carry

        lax.fori_loop(0, nvec // 8, _hist8, 0)

        for ch in range(16):
            acc = h8[pl.ds(ch * _NLANE, _NLANE)]
            h8[pl.ds(ch * _NLANE, _NLANE)] = zeros_i
            for l in range(1, _NLANE):
                off = l * 256 + ch * _NLANE
                acc = acc + h8[pl.ds(off, _NLANE)]
                h8[pl.ds(off, _NLANE)] = zeros_i
            s8[pl.ds(ch * _NLANE, _NLANE)] = lax.rev(
                jnp.cumsum(lax.rev(acc, (0,))), (0,))
        tot = plsc.load_gather(s8, [lane16])
        csum = lax.rev(jnp.cumsum(lax.rev(tot, (0,))), (0,))
        carry_v = csum - tot
        cnt_vec = zeros_i
        for ch in range(16):
            sufc = s8[pl.ds(ch * _NLANE, _NLANE)] + carry_v[ch]
            s8[pl.ds(ch * _NLANE, _NLANE)] = sufc
            cnt_vec = cnt_vec + (sufc >= k).astype(jnp.int32)
        cntge = jnp.sum(cnt_vec)
        bkt = cntge - 1
        kk = jnp.int32(k) - s8[pl.ds(bkt + 1, _NLANE)][0]

        def _compress(i, ptr):
            for u in range(4):
                j = i * 4 + u
                a = _abits(j)
                e = a >> 23
                orow[pl.ds(j * _NLANE, _NLANE)] = jnp.where(
                    e > bkt, 1.0, 0.0)
                m = e == bkt
                plsc.store_compressed(ca.at[pl.ds(ptr, _NLANE)], a, mask=m)
                plsc.store_compressed(pa.at[pl.ds(ptr, _NLANE)],
                                      j * _NLANE + lane, mask=m)
                ptr = ptr + plsc.all_reduce_population_count(m)[0]
            return ptr

        n_c = lax.fori_loop(0, nvec // 4, _compress, jnp.int32(0))

        bufs = ((ca, pa), (cb, pb))
        for ridx, (shift, width) in enumerate(_ROUNDS):
            src_a, src_p = bufs[ridx % 2]
            dst_a, dst_p = bufs[(ridx + 1) % 2]
            wmask = (1 << width) - 1
            for l in range(_NLANE):
                h4[pl.ds(l * _NLANE, _NLANE)] = zeros_i

            def _hist4(j, carry, src_a=src_a, n_c=n_c, shift=shift,
                       wmask=wmask):
                for u in range(2):
                    jj = j * 2 + u
                    av = src_a[pl.ds(jj * _NLANE, _NLANE)]
                    valid = (jj * _NLANE + lane) < n_c
                    nib = (av >> shift) & wmask
                    plsc.addupdate_scatter(h4, [lane16 + nib], ones_i,
                                           mask=valid)
                return carry

            lax.fori_loop(0, (n_c + 2 * _NLANE - 1) // (2 * _NLANE),
                          _hist4, 0)

            acc = h4[pl.ds(0, _NLANE)]
            for l in range(1, _NLANE):
                acc = acc + h4[pl.ds(l * _NLANE, _NLANE)]
            suf = lax.rev(jnp.cumsum(lax.rev(acc, (0,))), (0,))
            s4[pl.ds(0, _NLANE)] = suf
            nstar = jnp.sum((suf >= kk).astype(jnp.int32)) - 1
            kk = kk - s4[pl.ds(nstar + 1, _NLANE)][0]

            def _recomp(j, ptr, src_a=src_a, src_p=src_p, dst_a=dst_a,
                        dst_p=dst_p, n_c=n_c, shift=shift, wmask=wmask,
                        nstar=nstar):
                for u in range(2):
                    jj = j * 2 + u
                    av = src_a[pl.ds(jj * _NLANE, _NLANE)]
                    pv = src_p[pl.ds(jj * _NLANE, _NLANE)]
                    valid = (jj * _NLANE + lane) < n_c
                    nib = (av >> shift) & wmask
                    win = valid & (nib > nstar)
                    plsc.store_scatter(orow, [pv], ones_f, mask=win)
                    keep = valid & (nib == nstar)
                    plsc.store_compressed(dst_a.at[pl.ds(ptr, _NLANE)], av,
                                          mask=keep)
                    plsc.store_compressed(dst_p.at[pl.ds(ptr, _NLANE)], pv,
                                          mask=keep)
                    ptr = ptr + plsc.all_reduce_population_count(keep)[0]
                return ptr

            n_c = lax.fori_loop(0, (n_c + 2 * _NLANE - 1) // (2 * _NLANE),
                                _recomp, jnp.int32(0))

        tie_p = bufs[len(_ROUNDS) % 2][1]

        def _ties(j, carry, n_c=n_c):
            pv = tie_p[pl.ds(j * _NLANE, _NLANE)]
            valid = (j * _NLANE + lane) < n_c
            plsc.store_scatter(orow, [pv], ones_f, mask=valid)
            return carry

        lax.fori_loop(0, (n_c + (_NLANE - 1)) // _NLANE, _ties, 0)

        pltpu.sync_copy(orow, o_hbm.at[row])
        return carry

    lax.fori_loop(0, rows_per_worker, row_body, 0)


def kernel(x):
    B, N, C, L = x.shape
    k = int(L * _MASK_RATIO)
    M = B * N * C
    nw = 32
    rows_per_worker = M // nw
    nvec = L // _NLANE
    xf = x.reshape(M, L)

    sck = pl.kernel(
        functools.partial(_sc_body, k, rows_per_worker, nvec),
        out_type=jax.ShapeDtypeStruct((M, L), jnp.float32),
        mesh=plsc.VectorSubcoreMesh(core_axis_name="c", subcore_axis_name="s",
                                    num_cores=2, num_subcores=16),
        compiler_params=pltpu.CompilerParams(needs_layout_passes=False),
        scratch_types=[
            pltpu.VMEM((L,), jnp.float32),
            pltpu.VMEM((L,), jnp.float32),
            pltpu.VMEM((_NLANE * 256,), jnp.int32),
            pltpu.VMEM((256 + _NLANE,), jnp.int32),
            pltpu.VMEM((L + _NLANE,), jnp.int32),
            pltpu.VMEM((L + _NLANE,), jnp.int32),
            pltpu.VMEM((L + _NLANE,), jnp.int32),
            pltpu.VMEM((L + _NLANE,), jnp.int32),
            pltpu.VMEM((_NLANE * _NLANE,), jnp.int32),
            pltpu.VMEM((2 * _NLANE,), jnp.int32),
        ],
    )
    out = sck(xf)
    return out.reshape(B, N, C, L)


# speedup vs baseline: 1.6558x; 1.6558x over previous
import functools

import jax
import jax.numpy as jnp
from jax import lax
from jax.experimental import pallas as pl
from jax.experimental.pallas import tpu as pltpu
from jax.experimental.pallas import tpu_sc as plsc

_MASK_RATIO = 0.25
_NLANE = 16
_ROUNDS = ((19, 4), (15, 4), (11, 4), (7, 4), (3, 4), (0, 3))


def _tree_sum(vs):
    while len(vs) > 1:
        vs = [vs[i] + vs[i + 1] for i in range(0, len(vs) - 1, 2)] + (
            [vs[-1]] if len(vs) % 2 else [])
    return vs[0]


def _suffix(v):
    return lax.rev(jnp.cumsum(lax.rev(v, (0,))), (0,))


def _sc_body(k, rows_per_worker, nvec, x_hbm, o_hbm, xrow, orow, h8, s8,
             ca, pa, cb, pb, h4, s4):
    c = lax.axis_index("c")
    s = lax.axis_index("s")
    wid = s * 2 + c
    lane = lax.iota(jnp.int32, _NLANE)
    lane256 = lane * 256
    lane16 = lane * _NLANE
    ones_i = jnp.ones((_NLANE,), jnp.int32)
    zeros_i = jnp.zeros((_NLANE,), jnp.int32)
    ones_f = jnp.ones((_NLANE,), jnp.float32)

    def _abits(i):
        v = xrow[pl.ds(i * _NLANE, _NLANE)]
        return lax.bitcast_convert_type(v, jnp.int32) & 0x7FFFFFFF

    def _zero_h8(i, carry):
        h8[pl.ds(i * _NLANE, _NLANE)] = zeros_i
        return carry

    lax.fori_loop(0, 256, _zero_h8, 0)
    s8[pl.ds(256, _NLANE)] = zeros_i
    s4[pl.ds(_NLANE, _NLANE)] = zeros_i

    def row_body(r, carry):
        row = wid * rows_per_worker + r
        pltpu.sync_copy(x_hbm.at[row], xrow)

        @plsc.parallel_loop(0, nvec, step=1, unroll=8)
        def _hist8(i):
            a = _abits(i)
            plsc.addupdate_scatter(h8, [lane256 + (a >> 23)], ones_i)

        for ch in range(16):
            vs = [h8[pl.ds(l * 256 + ch * _NLANE, _NLANE)]
                  for l in range(_NLANE)]
            for l in range(_NLANE):
                h8[pl.ds(l * 256 + ch * _NLANE, _NLANE)] = zeros_i
            s8[pl.ds(ch * _NLANE, _NLANE)] = _suffix(_tree_sum(vs))
        tot = plsc.load_gather(s8, [lane16])
        carry_v = _suffix(tot) - tot
        cnt_parts = []
        for ch in range(16):
            sufc = s8[pl.ds(ch * _NLANE, _NLANE)] + carry_v[ch]
            s8[pl.ds(ch * _NLANE, _NLANE)] = sufc
            cnt_parts.append((sufc >= k).astype(jnp.int32))
        cntge = jnp.sum(_tree_sum(cnt_parts))
        bkt = cntge - 1
        kk = jnp.int32(k) - s8[pl.ds(bkt + 1, _NLANE)][0]

        @plsc.parallel_loop(0, nvec, step=1, unroll=4, carry=zeros_i)
        def _compress(i, ptrv):
            a = _abits(i)
            e = a >> 23
            orow[pl.ds(i * _NLANE, _NLANE)] = jnp.where(e > bkt, 1.0, 0.0)
            m = e == bkt
            idx = ptrv + jnp.cumsum(m.astype(jnp.int32)) - 1
            plsc.store_scatter(ca, [idx], a, mask=m)
            plsc.store_scatter(pa, [idx], i * _NLANE + lane, mask=m)
            return ptrv + plsc.all_reduce_population_count(m)

        n_c = _compress[0]

        bufs = ((ca, pa), (cb, pb))
        for ridx, (shift, width) in enumerate(_ROUNDS):
            src_a, src_p = bufs[ridx % 2]
            dst_a, dst_p = bufs[(ridx + 1) % 2]
            wmask = (1 << width) - 1
            for l in range(_NLANE):
                h4[pl.ds(l * _NLANE, _NLANE)] = zeros_i
            nv = (n_c + _NLANE - 1) // _NLANE

            @plsc.parallel_loop(0, nv, step=1, unroll=2)
            def _hist4(j, src_a=src_a, n_c=n_c, shift=shift, wmask=wmask):
                av = src_a[pl.ds(j * _NLANE, _NLANE)]
                valid = (j * _NLANE + lane) < n_c
                nib = (av >> shift) & wmask
                plsc.addupdate_scatter(h4, [lane16 + nib], ones_i, mask=valid)

            acc = _tree_sum([h4[pl.ds(l * _NLANE, _NLANE)]
                             for l in range(_NLANE)])
            suf = _suffix(acc)
            s4[pl.ds(0, _NLANE)] = suf
            nstar = jnp.sum((suf >= kk).astype(jnp.int32)) - 1
            kk = kk - s4[pl.ds(nstar + 1, _NLANE)][0]

            @plsc.parallel_loop(0, nv, step=1, unroll=2, carry=zeros_i)
            def _recomp(j, ptrv, src_a=src_a, src_p=src_p, dst_a=dst_a,
                        dst_p=dst_p, n_c=n_c, shift=shift, wmask=wmask,
                        nstar=nstar):
                av = src_a[pl.ds(j * _NLANE, _NLANE)]
                pv = src_p[pl.ds(j * _NLANE, _NLANE)]
                valid = (j * _NLANE + lane) < n_c
                nib = (av >> shift) & wmask
                win = valid & (nib > nstar)
                plsc.store_scatter(orow, [pv], ones_f, mask=win)
                keep = valid & (nib == nstar)
                idx = ptrv + jnp.cumsum(keep.astype(jnp.int32)) - 1
                plsc.store_scatter(dst_a, [idx], av, mask=keep)
                plsc.store_scatter(dst_p, [idx], pv, mask=keep)
                return ptrv + plsc.all_reduce_population_count(keep)

            n_c = _recomp[0]

        tie_p = bufs[len(_ROUNDS) % 2][1]

        def _ties(j, carry, n_c=n_c):
            pv = tie_p[pl.ds(j * _NLANE, _NLANE)]
            valid = (j * _NLANE + lane) < n_c
            plsc.store_scatter(orow, [pv], ones_f, mask=valid)
            return carry

        lax.fori_loop(0, (n_c + (_NLANE - 1)) // _NLANE, _ties, 0)

        pltpu.sync_copy(orow, o_hbm.at[row])
        return carry

    lax.fori_loop(0, rows_per_worker, row_body, 0)


def kernel(x):
    B, N, C, L = x.shape
    k = int(L * _MASK_RATIO)
    M = B * N * C
    nw = 32
    rows_per_worker = M // nw
    nvec = L // _NLANE
    xf = x.reshape(M, L)

    sck = pl.kernel(
        functools.partial(_sc_body, k, rows_per_worker, nvec),
        out_type=jax.ShapeDtypeStruct((M, L), jnp.float32),
        mesh=plsc.VectorSubcoreMesh(core_axis_name="c", subcore_axis_name="s",
                                    num_cores=2, num_subcores=16),
        compiler_params=pltpu.CompilerParams(needs_layout_passes=False),
        scratch_types=[
            pltpu.VMEM((L,), jnp.float32),
            pltpu.VMEM((L,), jnp.float32),
            pltpu.VMEM((_NLANE * 256,), jnp.int32),
            pltpu.VMEM((256 + _NLANE,), jnp.int32),
            pltpu.VMEM((L + _NLANE,), jnp.int32),
            pltpu.VMEM((L + _NLANE,), jnp.int32),
            pltpu.VMEM((L + _NLANE,), jnp.int32),
            pltpu.VMEM((L + _NLANE,), jnp.int32),
            pltpu.VMEM((_NLANE * _NLANE,), jnp.int32),
            pltpu.VMEM((2 * _NLANE,), jnp.int32),
        ],
    )
    out = sck(xf)
    return out.reshape(B, N, C, L)


# speedup vs baseline: 2.2048x; 1.3316x over previous
import functools

import jax
import jax.numpy as jnp
from jax import lax
from jax.experimental import pallas as pl
from jax.experimental.pallas import tpu as pltpu
from jax.experimental.pallas import tpu_sc as plsc

_MASK_RATIO = 0.25
_NLANE = 16
_ROUNDS = ((19, 4), (15, 4), (11, 4), (7, 4), (3, 4), (0, 3))


def _tree_sum(vs):
    while len(vs) > 1:
        vs = [vs[i] + vs[i + 1] for i in range(0, len(vs) - 1, 2)] + (
            [vs[-1]] if len(vs) % 2 else [])
    return vs[0]


def _suffix(v):
    return lax.rev(jnp.cumsum(lax.rev(v, (0,))), (0,))


def _sc_body(k, rows_per_worker, nvec, x_hbm, o_hbm, xr0, xr1, or0, or1,
             h8, s8, ca, pa, cb, pb, h4, s4, in0, in1, out0, out1):
    c = lax.axis_index("c")
    s = lax.axis_index("s")
    wid = s * 2 + c
    lane = lax.iota(jnp.int32, _NLANE)
    lane273 = lane * 273
    lane19 = lane * 19
    lane16 = lane * _NLANE
    ones_i = jnp.ones((_NLANE,), jnp.int32)
    zeros_i = jnp.zeros((_NLANE,), jnp.int32)
    ones_f = jnp.ones((_NLANE,), jnp.float32)

    def _zero_h8(i, carry):
        h8[pl.ds(i * _NLANE, _NLANE)] = zeros_i
        return carry

    lax.fori_loop(0, (_NLANE * 273 + _NLANE - 1) // _NLANE, _zero_h8, 0)
    s8[pl.ds(256, _NLANE)] = zeros_i
    s4[pl.ds(_NLANE, _NLANE)] = zeros_i

    base = wid * rows_per_worker
    npairs = rows_per_worker // 2

    def _process(h, row, xrow, orow, in_sem, out_sem):
        def _abits(i):
            v = xrow[pl.ds(i * _NLANE, _NLANE)]
            return lax.bitcast_convert_type(v, jnp.int32) & 0x7FFFFFFF

        @plsc.parallel_loop(0, nvec, step=1, unroll=8)
        def _hist8(i):
            a = _abits(i)
            plsc.addupdate_scatter(h8, [lane273 + (a >> 23)], ones_i)

        for ch in range(16):
            vs = [h8[pl.ds(l * 273 + ch * _NLANE, _NLANE)]
                  for l in range(_NLANE)]
            for l in range(_NLANE):
                h8[pl.ds(l * 273 + ch * _NLANE, _NLANE)] = zeros_i
            s8[pl.ds(ch * _NLANE, _NLANE)] = _suffix(_tree_sum(vs))
        tot = plsc.load_gather(s8, [lane16])
        carry_v = _suffix(tot) - tot
        cnt_parts = []
        for ch in range(16):
            sufc = s8[pl.ds(ch * _NLANE, _NLANE)] + carry_v[ch]
            s8[pl.ds(ch * _NLANE, _NLANE)] = sufc
            cnt_parts.append((sufc >= k).astype(jnp.int32))
        cntge = jnp.sum(_tree_sum(cnt_parts))
        bkt = cntge - 1
        kk = jnp.int32(k) - s8[pl.ds(bkt + 1, _NLANE)][0]

        @pl.when(h > 0)
        def _drain():
            pltpu.make_async_copy(orow, o_hbm.at[row], out_sem).wait()

        @plsc.parallel_loop(0, nvec, step=1, unroll=4, carry=zeros_i)
        def _compress(i, ptrv):
            a = _abits(i)
            e = a >> 23
            orow[pl.ds(i * _NLANE, _NLANE)] = jnp.where(e > bkt, 1.0, 0.0)
            m = e == bkt
            idx = ptrv + jnp.cumsum(m.astype(jnp.int32)) - 1
            plsc.store_scatter(ca, [idx], a, mask=m)
            plsc.store_scatter(pa, [idx], i * _NLANE + lane, mask=m)
            return ptrv + plsc.all_reduce_population_count(m)

        n_c = _compress[0]

        @pl.when(h + 1 < npairs)
        def _prefetch():
            pltpu.async_copy(x_hbm.at[row + 2], xrow, in_sem)

        bufs = ((ca, pa), (cb, pb))
        for ridx, (shift, width) in enumerate(_ROUNDS):
            src_a, src_p = bufs[ridx % 2]
            dst_a, dst_p = bufs[(ridx + 1) % 2]
            wmask = (1 << width) - 1
            for l in range(_NLANE):
                h4[pl.ds(l * 19, _NLANE)] = zeros_i
            nv = (n_c + _NLANE - 1) // _NLANE

            @plsc.parallel_loop(0, nv, step=1, unroll=2)
            def _hist4(j, src_a=src_a, n_c=n_c, shift=shift, wmask=wmask):
                av = src_a[pl.ds(j * _NLANE, _NLANE)]
                valid = (j * _NLANE + lane) < n_c
                nib = (av >> shift) & wmask
                plsc.addupdate_scatter(h4, [lane19 + nib], ones_i,
                                       mask=valid)

            acc = _tree_sum([h4[pl.ds(l * 19, _NLANE)]
                             for l in range(_NLANE)])
            suf = _suffix(acc)
            s4[pl.ds(0, _NLANE)] = suf
            nstar = jnp.sum((suf >= kk).astype(jnp.int32)) - 1
            kk = kk - s4[pl.ds(nstar + 1, _NLANE)][0]

            @plsc.parallel_loop(0, nv, step=1, unroll=2, carry=zeros_i)
            def _recomp(j, ptrv, src_a=src_a, src_p=src_p, dst_a=dst_a,
                        dst_p=dst_p, n_c=n_c, shift=shift, wmask=wmask,
                        nstar=nstar):
                av = src_a[pl.ds(j * _NLANE, _NLANE)]
                pv = src_p[pl.ds(j * _NLANE, _NLANE)]
                valid = (j * _NLANE + lane) < n_c
                nib = (av >> shift) & wmask
                win = valid & (nib > nstar)
                plsc.store_scatter(orow, [pv], ones_f, mask=win)
                keep = valid & (nib == nstar)
                idx = ptrv + jnp.cumsum(keep.astype(jnp.int32)) - 1
                plsc.store_scatter(dst_a, [idx], av, mask=keep)
                plsc.store_scatter(dst_p, [idx], pv, mask=keep)
                return ptrv + plsc.all_reduce_population_count(keep)

            n_c = _recomp[0]

        tie_p = bufs[len(_ROUNDS) % 2][1]

        def _ties(j, carry, n_c=n_c):
            pv = tie_p[pl.ds(j * _NLANE, _NLANE)]
            valid = (j * _NLANE + lane) < n_c
            plsc.store_scatter(orow, [pv], ones_f, mask=valid)
            return carry

        lax.fori_loop(0, (n_c + (_NLANE - 1)) // _NLANE, _ties, 0)

        pltpu.async_copy(orow, o_hbm.at[row], out_sem)

    pltpu.async_copy(x_hbm.at[base], xr0, in0)
    pltpu.async_copy(x_hbm.at[base + 1], xr1, in1)

    def pair_body(h, carry):
        r0 = base + 2 * h
        pltpu.make_async_copy(x_hbm.at[r0], xr0, in0).wait()
        _process(h, r0, xr0, or0, in0, out0)
        pltpu.make_async_copy(x_hbm.at[r0 + 1], xr1, in1).wait()
        _process(h, r0 + 1, xr1, or1, in1, out1)
        return carry

    lax.fori_loop(0, npairs, pair_body, 0)

    last = base + rows_per_worker - 2
    pltpu.make_async_copy(or0, o_hbm.at[last], out0).wait()
    pltpu.make_async_copy(or1, o_hbm.at[last + 1], out1).wait()


def kernel(x):
    B, N, C, L = x.shape
    k = int(L * _MASK_RATIO)
    M = B * N * C
    nw = 32
    rows_per_worker = M // nw
    nvec = L // _NLANE
    xf = x.reshape(M, L)

    sck = pl.kernel(
        functools.partial(_sc_body, k, rows_per_worker, nvec),
        out_type=jax.ShapeDtypeStruct((M, L), jnp.float32),
        mesh=plsc.VectorSubcoreMesh(core_axis_name="c", subcore_axis_name="s",
                                    num_cores=2, num_subcores=16),
        compiler_params=pltpu.CompilerParams(needs_layout_passes=False),
        scratch_types=[
            pltpu.VMEM((L,), jnp.float32),
            pltpu.VMEM((L,), jnp.float32),
            pltpu.VMEM((L,), jnp.float32),
            pltpu.VMEM((L,), jnp.float32),
            pltpu.VMEM((_NLANE * 273,), jnp.int32),
            pltpu.VMEM((256 + _NLANE,), jnp.int32),
            pltpu.VMEM((L + _NLANE,), jnp.int32),
            pltpu.VMEM((L + _NLANE,), jnp.int32),
            pltpu.VMEM((L + _NLANE,), jnp.int32),
            pltpu.VMEM((L + _NLANE,), jnp.int32),
            pltpu.VMEM((_NLANE * 19 + _NLANE,), jnp.int32),
            pltpu.VMEM((2 * _NLANE,), jnp.int32),
            pltpu.SemaphoreType.DMA,
            pltpu.SemaphoreType.DMA,
            pltpu.SemaphoreType.DMA,
            pltpu.SemaphoreType.DMA,
        ],
    )
    out = sck(xf)
    return out.reshape(B, N, C, L)


# speedup vs baseline: 3.2008x; 1.4517x over previous
import functools

import jax
import jax.numpy as jnp
from jax import lax
from jax.experimental import pallas as pl
from jax.experimental.pallas import tpu as pltpu
from jax.experimental.pallas import tpu_sc as plsc

_MASK_RATIO = 0.25
_NLANE = 16
_ROUNDS = ((19, 4), (15, 4), (11, 4), (7, 4), (3, 4), (0, 3))


def _tree_sum(vs):
    while len(vs) > 1:
        vs = [vs[i] + vs[i + 1] for i in range(0, len(vs) - 1, 2)] + (
            [vs[-1]] if len(vs) % 2 else [])
    return vs[0]


def _suffix(v):
    return lax.rev(jnp.cumsum(lax.rev(v, (0,))), (0,))


def _sc_body(k, rows_per_worker, nvec, x_hbm, o_hbm, xr0, xr1, or0, or1,
             h8, s8, ca, pa, cb, pb, h4, s4, in0, in1, out0, out1):
    c = lax.axis_index("c")
    s = lax.axis_index("s")
    wid = s * 2 + c
    lane = lax.iota(jnp.int32, _NLANE)
    lane273 = lane * 273
    lane19 = lane * 19
    lane16 = lane * _NLANE
    ones_i = jnp.ones((_NLANE,), jnp.int32)
    zeros_i = jnp.zeros((_NLANE,), jnp.int32)
    ones_f = jnp.ones((_NLANE,), jnp.float32)

    def _zero_h8(i, carry):
        h8[pl.ds(i * _NLANE, _NLANE)] = zeros_i
        return carry

    lax.fori_loop(0, (_NLANE * 273 + _NLANE - 1) // _NLANE, _zero_h8, 0)
    s8[pl.ds(256, _NLANE)] = zeros_i
    s4[pl.ds(_NLANE, _NLANE)] = zeros_i

    base = wid * rows_per_worker
    npairs = rows_per_worker // 2

    def _process(h, row, xrow, orow, in_sem, out_sem):
        def _abits(i):
            v = xrow[pl.ds(i * _NLANE, _NLANE)]
            return lax.bitcast_convert_type(v, jnp.int32) & 0x7FFFFFFF

        @plsc.parallel_loop(0, nvec, step=1, unroll=8)
        def _hist8(i):
            a = _abits(i)
            plsc.addupdate_scatter(h8, [lane273 + (a >> 23)], ones_i)

        for ch in range(16):
            vs = [h8[pl.ds(l * 273 + ch * _NLANE, _NLANE)]
                  for l in range(_NLANE)]
            for l in range(_NLANE):
                h8[pl.ds(l * 273 + ch * _NLANE, _NLANE)] = zeros_i
            s8[pl.ds(ch * _NLANE, _NLANE)] = _suffix(_tree_sum(vs))
        tot = plsc.load_gather(s8, [lane16])
        carry_v = _suffix(tot) - tot
        cnt_parts = []
        for ch in range(16):
            sufc = s8[pl.ds(ch * _NLANE, _NLANE)] + carry_v[ch]
            s8[pl.ds(ch * _NLANE, _NLANE)] = sufc
            cnt_parts.append((sufc >= k).astype(jnp.int32))
        cntge = jnp.sum(_tree_sum(cnt_parts))
        bkt = cntge - 1
        kk = jnp.int32(k) - s8[pl.ds(bkt + 1, _NLANE)][0]

        @pl.when(h > 0)
        def _drain():
            pltpu.make_async_copy(orow, o_hbm.at[row], out_sem).wait()

        @plsc.parallel_loop(0, nvec, step=1, unroll=4, carry=zeros_i)
        def _compress(i, ptrv):
            a = _abits(i)
            e = a >> 23
            orow[pl.ds(i * _NLANE, _NLANE)] = jnp.where(e > bkt, 1.0, 0.0)
            m = e == bkt
            idx = ptrv + jnp.cumsum(m.astype(jnp.int32)) - 1
            plsc.store_scatter(ca, [idx], a, mask=m)
            plsc.store_scatter(pa, [idx], i * _NLANE + lane, mask=m)
            return ptrv + plsc.all_reduce_population_count(m)

        n_c = _compress[0]

        @pl.when(h + 1 < npairs)
        def _prefetch():
            pltpu.async_copy(x_hbm.at[row + 2], xrow, in_sem)

        shift0 = _ROUNDS[0][0]
        for l in range(_NLANE):
            h4[pl.ds(l * 19, _NLANE)] = zeros_i
        nv0 = (n_c + _NLANE - 1) // _NLANE

        @plsc.parallel_loop(0, nv0, step=1, unroll=2)
        def _hist4(j, n_c=n_c):
            av = ca[pl.ds(j * _NLANE, _NLANE)]
            valid = (j * _NLANE + lane) < n_c
            nib = (av >> shift0) & 0xF
            plsc.addupdate_scatter(h4, [lane19 + nib], ones_i, mask=valid)

        acc = _tree_sum([h4[pl.ds(l * 19, _NLANE)] for l in range(_NLANE)])
        suf = _suffix(acc)
        s4[pl.ds(0, _NLANE)] = suf
        nstar = jnp.sum((suf >= kk).astype(jnp.int32)) - 1
        kk = kk - s4[pl.ds(nstar + 1, _NLANE)][0]

        @plsc.parallel_loop(0, nv0, step=1, unroll=2, carry=zeros_i)
        def _recomp0(j, ptrv, n_c=n_c, nstar=nstar):
            av = ca[pl.ds(j * _NLANE, _NLANE)]
            pv = pa[pl.ds(j * _NLANE, _NLANE)]
            valid = (j * _NLANE + lane) < n_c
            nib = (av >> shift0) & 0xF
            win = valid & (nib > nstar)
            plsc.store_scatter(orow, [pv], ones_f, mask=win)
            keep = valid & (nib == nstar)
            idx = ptrv + jnp.cumsum(keep.astype(jnp.int32)) - 1
            plsc.store_scatter(cb, [idx], av, mask=keep)
            plsc.store_scatter(pb, [idx], pv, mask=keep)
            return ptrv + plsc.all_reduce_population_count(keep)

        n_c = _recomp0[0]

        def _round_inplace(args, shift, width):
            n_c, kk = args
            wmask = (1 << width) - 1
            for l in range(_NLANE):
                h4[pl.ds(l * 19, _NLANE)] = zeros_i
            nv = (n_c + _NLANE - 1) // _NLANE

            def _h(j, carry):
                av = cb[pl.ds(j * _NLANE, _NLANE)]
                valid = (j * _NLANE + lane) < n_c
                nib = (av >> shift) & wmask
                plsc.addupdate_scatter(h4, [lane19 + nib], ones_i,
                                       mask=valid)
                return carry

            lax.fori_loop(0, nv, _h, 0)
            acc = _tree_sum([h4[pl.ds(l * 19, _NLANE)]
                             for l in range(_NLANE)])
            suf = _suffix(acc)
            s4[pl.ds(0, _NLANE)] = suf
            nstar = jnp.sum((suf >= kk).astype(jnp.int32)) - 1
            kk2 = kk - s4[pl.ds(nstar + 1, _NLANE)][0]

            def _rc(j, ptr):
                av = cb[pl.ds(j * _NLANE, _NLANE)]
                pv = pb[pl.ds(j * _NLANE, _NLANE)]
                valid = (j * _NLANE + lane) < n_c
                nib = (av >> shift) & wmask
                win = valid & (nib > nstar)
                plsc.store_scatter(orow, [pv], ones_f, mask=win)
                keep = valid & (nib == nstar)
                idx = ptr + jnp.cumsum(keep.astype(jnp.int32)) - 1
                plsc.store_scatter(cb, [idx], av, mask=keep)
                plsc.store_scatter(pb, [idx], pv, mask=keep)
                return ptr + jnp.sum(keep.astype(jnp.int32))

            n_c2 = lax.fori_loop(0, nv, _rc, jnp.int32(0))
            return n_c2, kk2

        state = (n_c, kk)
        for shift, width in _ROUNDS[1:]:
            state = lax.cond(
                state[0] > _NLANE,
                functools.partial(_round_inplace, shift=shift, width=width),
                lambda args: args,
                state)
        n_c, kk = state

        def _fin_small(args):
            n_c, kk = args
            av = cb[pl.ds(0, _NLANE)]
            pv = pb[pl.ds(0, _NLANE)]
            valid = lane < n_c
            srt = jnp.sort(jnp.where(valid, av, -1))
            s4[pl.ds(0, _NLANE)] = srt
            thr = s4[pl.ds(_NLANE - kk, _NLANE)][0]
            win = valid & (av >= thr)
            plsc.store_scatter(orow, [pv], ones_f, mask=win)
            return 0

        def _fin_ties(args):
            n_c, _ = args

            def _ties(j, carry):
                pv = pb[pl.ds(j * _NLANE, _NLANE)]
                valid = (j * _NLANE + lane) < n_c
                plsc.store_scatter(orow, [pv], ones_f, mask=valid)
                return carry

            lax.fori_loop(0, (n_c + (_NLANE - 1)) // _NLANE, _ties, 0)
            return 0

        lax.cond(n_c <= _NLANE, _fin_small, _fin_ties, (n_c, kk))

        pltpu.async_copy(orow, o_hbm.at[row], out_sem)

    pltpu.async_copy(x_hbm.at[base], xr0, in0)
    pltpu.async_copy(x_hbm.at[base + 1], xr1, in1)

    def pair_body(h, carry):
        r0 = base + 2 * h
        pltpu.make_async_copy(x_hbm.at[r0], xr0, in0).wait()
        _process(h, r0, xr0, or0, in0, out0)
        pltpu.make_async_copy(x_hbm.at[r0 + 1], xr1, in1).wait()
        _process(h, r0 + 1, xr1, or1, in1, out1)
        return carry

    lax.fori_loop(0, npairs, pair_body, 0)

    last = base + rows_per_worker - 2
    pltpu.make_async_copy(or0, o_hbm.at[last], out0).wait()
    pltpu.make_async_copy(or1, o_hbm.at[last + 1], out1).wait()


def kernel(x):
    B, N, C, L = x.shape
    k = int(L * _MASK_RATIO)
    M = B * N * C
    nw = 32
    rows_per_worker = M // nw
    nvec = L // _NLANE
    xf = x.reshape(M, L)

    sck = pl.kernel(
        functools.partial(_sc_body, k, rows_per_worker, nvec),
        out_type=jax.ShapeDtypeStruct((M, L), jnp.float32),
        mesh=plsc.VectorSubcoreMesh(core_axis_name="c", subcore_axis_name="s",
                                    num_cores=2, num_subcores=16),
        compiler_params=pltpu.CompilerParams(needs_layout_passes=False),
        scratch_types=[
            pltpu.VMEM((L,), jnp.float32),
            pltpu.VMEM((L,), jnp.float32),
            pltpu.VMEM((L,), jnp.float32),
            pltpu.VMEM((L,), jnp.float32),
            pltpu.VMEM((_NLANE * 273,), jnp.int32),
            pltpu.VMEM((256 + _NLANE,), jnp.int32),
            pltpu.VMEM((L + _NLANE,), jnp.int32),
            pltpu.VMEM((L + _NLANE,), jnp.int32),
            pltpu.VMEM((L + _NLANE,), jnp.int32),
            pltpu.VMEM((L + _NLANE,), jnp.int32),
            pltpu.VMEM((_NLANE * 19 + _NLANE,), jnp.int32),
            pltpu.VMEM((2 * _NLANE,), jnp.int32),
            pltpu.SemaphoreType.DMA,
            pltpu.SemaphoreType.DMA,
            pltpu.SemaphoreType.DMA,
            pltpu.SemaphoreType.DMA,
        ],
    )
    out = sck(xf)
    return out.reshape(B, N, C, L)


# speedup vs baseline: 3.2085x; 1.0024x over previous
import functools

import jax
import jax.numpy as jnp
from jax import lax
from jax.experimental import pallas as pl
from jax.experimental.pallas import tpu as pltpu
from jax.experimental.pallas import tpu_sc as plsc

_MASK_RATIO = 0.25
_NLANE = 16
_ROUNDS = ((19, 4), (15, 4), (11, 4), (7, 4), (3, 4), (0, 3))


def _tree_sum(vs):
    while len(vs) > 1:
        vs = [vs[i] + vs[i + 1] for i in range(0, len(vs) - 1, 2)] + (
            [vs[-1]] if len(vs) % 2 else [])
    return vs[0]


def _suffix(v):
    return lax.rev(jnp.cumsum(lax.rev(v, (0,))), (0,))


def _sc_body(k, rows_per_worker, nvec, x_hbm, o_hbm, xr0, xr1, or0, or1,
             h8, s8, ca, pa, cb, pb, h4, s4, in0, in1, out0, out1):
    c = lax.axis_index("c")
    s = lax.axis_index("s")
    wid = s * 2 + c
    lane = lax.iota(jnp.int32, _NLANE)
    lane273 = lane * 273
    lane19 = lane * 19
    lane16 = lane * _NLANE
    ones_i = jnp.ones((_NLANE,), jnp.int32)
    zeros_i = jnp.zeros((_NLANE,), jnp.int32)
    ones_f = jnp.ones((_NLANE,), jnp.float32)

    def _zero_h8(i, carry):
        h8[pl.ds(i * _NLANE, _NLANE)] = zeros_i
        return carry

    lax.fori_loop(0, (_NLANE * 273 + _NLANE - 1) // _NLANE, _zero_h8, 0)
    s8[pl.ds(256, _NLANE)] = zeros_i
    s4[pl.ds(_NLANE, _NLANE)] = zeros_i

    base = wid * rows_per_worker
    npairs = rows_per_worker // 2

    def _process(h, row, xrow, orow, in_sem, out_sem):
        def _abits(i):
            v = xrow[pl.ds(i * _NLANE, _NLANE)]
            return lax.bitcast_convert_type(v, jnp.int32) & 0x7FFFFFFF

        @plsc.parallel_loop(0, nvec, step=1, unroll=8)
        def _hist8(i):
            a = _abits(i)
            plsc.addupdate_scatter(h8, [lane273 + (a >> 23)], ones_i)

        for ch in range(16):
            vs = [h8[pl.ds(l * 273 + ch * _NLANE, _NLANE)]
                  for l in range(_NLANE)]
            for l in range(_NLANE):
                h8[pl.ds(l * 273 + ch * _NLANE, _NLANE)] = zeros_i
            s8[pl.ds(ch * _NLANE, _NLANE)] = _suffix(_tree_sum(vs))
        tot = plsc.load_gather(s8, [lane16])
        carry_v = _suffix(tot) - tot
        cnt_parts = []
        for ch in range(16):
            sufc = s8[pl.ds(ch * _NLANE, _NLANE)] + carry_v[ch]
            s8[pl.ds(ch * _NLANE, _NLANE)] = sufc
            cnt_parts.append((sufc >= k).astype(jnp.int32))
        cntge = jnp.sum(_tree_sum(cnt_parts))
        bkt = cntge - 1
        kk = jnp.int32(k) - s8[pl.ds(bkt + 1, _NLANE)][0]

        @pl.when(h > 0)
        def _drain():
            pltpu.make_async_copy(orow, o_hbm.at[row], out_sem).wait()

        @plsc.parallel_loop(0, nvec, step=1, unroll=4, carry=zeros_i)
        def _compress(i, ptrv):
            a = _abits(i)
            e = a >> 23
            orow[pl.ds(i * _NLANE, _NLANE)] = jnp.where(e > bkt, 1.0, 0.0)
            m = e == bkt
            idx = ptrv + jnp.cumsum(m.astype(jnp.int32)) - 1
            plsc.store_scatter(ca, [idx], a, mask=m)
            plsc.store_scatter(pa, [idx], i * _NLANE + lane, mask=m)
            return ptrv + plsc.all_reduce_population_count(m)

        n_c = _compress[0]

        @pl.when(h + 1 < npairs)
        def _prefetch():
            pltpu.async_copy(x_hbm.at[row + 2], xrow, in_sem)

        shift0 = _ROUNDS[0][0]
        for l in range(_NLANE):
            h4[pl.ds(l * 19, _NLANE)] = zeros_i
        nv0 = (n_c + _NLANE - 1) // _NLANE

        @plsc.parallel_loop(0, nv0, step=1, unroll=2)
        def _hist4(j, n_c=n_c):
            av = ca[pl.ds(j * _NLANE, _NLANE)]
            valid = (j * _NLANE + lane) < n_c
            nib = (av >> shift0) & 0xF
            plsc.addupdate_scatter(h4, [lane19 + nib], ones_i, mask=valid)

        acc = _tree_sum([h4[pl.ds(l * 19, _NLANE)] for l in range(_NLANE)])
        suf = _suffix(acc)
        s4[pl.ds(0, _NLANE)] = suf
        nstar = jnp.sum((suf >= kk).astype(jnp.int32)) - 1
        kk = kk - s4[pl.ds(nstar + 1, _NLANE)][0]

        @plsc.parallel_loop(0, nv0, step=1, unroll=2, carry=zeros_i)
        def _recomp0(j, ptrv, n_c=n_c, nstar=nstar):
            av = ca[pl.ds(j * _NLANE, _NLANE)]
            pv = pa[pl.ds(j * _NLANE, _NLANE)]
            valid = (j * _NLANE + lane) < n_c
            nib = (av >> shift0) & 0xF
            win = valid & (nib > nstar)
            plsc.store_scatter(orow, [pv], ones_f, mask=win)
            keep = valid & (nib == nstar)
            idx = ptrv + jnp.cumsum(keep.astype(jnp.int32)) - 1
            plsc.store_scatter(cb, [idx], av, mask=keep)
            plsc.store_scatter(pb, [idx], pv, mask=keep)
            return ptrv + plsc.all_reduce_population_count(keep)

        n_c = _recomp0[0]

        def _round_inplace(args, shift, width):
            n_c, kk = args
            wmask = (1 << width) - 1
            for l in range(_NLANE):
                h4[pl.ds(l * 19, _NLANE)] = zeros_i
            nv = (n_c + _NLANE - 1) // _NLANE

            def _h(j, carry):
                av = cb[pl.ds(j * _NLANE, _NLANE)]
                valid = (j * _NLANE + lane) < n_c
                nib = (av >> shift) & wmask
                plsc.addupdate_scatter(h4, [lane19 + nib], ones_i,
                                       mask=valid)
                return carry

            lax.fori_loop(0, nv, _h, 0)
            acc = _tree_sum([h4[pl.ds(l * 19, _NLANE)]
                             for l in range(_NLANE)])
            suf = _suffix(acc)
            s4[pl.ds(0, _NLANE)] = suf
            nstar = jnp.sum((suf >= kk).astype(jnp.int32)) - 1
            kk2 = kk - s4[pl.ds(nstar + 1, _NLANE)][0]

            def _rc(j, ptr):
                av = cb[pl.ds(j * _NLANE, _NLANE)]
                pv = pb[pl.ds(j * _NLANE, _NLANE)]
                valid = (j * _NLANE + lane) < n_c
                nib = (av >> shift) & wmask
                win = valid & (nib > nstar)
                plsc.store_scatter(orow, [pv], ones_f, mask=win)
                keep = valid & (nib == nstar)
                idx = ptr + jnp.cumsum(keep.astype(jnp.int32)) - 1
                plsc.store_scatter(cb, [idx], av, mask=keep)
                plsc.store_scatter(pb, [idx], pv, mask=keep)
                return ptr + jnp.sum(keep.astype(jnp.int32))

            n_c2 = lax.fori_loop(0, nv, _rc, jnp.int32(0))
            return n_c2, kk2

        state = (n_c, kk)
        for shift, width in _ROUNDS[1:]:
            state = lax.cond(
                state[0] > _NLANE,
                functools.partial(_round_inplace, shift=shift, width=width),
                lambda args: args,
                state)
        n_c, kk = state

        def _fin_small(args):
            n_c, kk = args
            av = cb[pl.ds(0, _NLANE)]
            pv = pb[pl.ds(0, _NLANE)]
            valid = lane < n_c
            srt = jnp.sort(jnp.where(valid, av, 0))
            s4[pl.ds(0, _NLANE)] = srt
            thr = s4[pl.ds(_NLANE - kk, _NLANE)][0]
            win = valid & (av >= thr)
            plsc.store_scatter(orow, [pv], ones_f, mask=win)
            return 0

        def _fin_ties(args):
            n_c, _ = args

            def _ties(j, carry):
                pv = pb[pl.ds(j * _NLANE, _NLANE)]
                valid = (j * _NLANE + lane) < n_c
                plsc.store_scatter(orow, [pv], ones_f, mask=valid)
                return carry

            lax.fori_loop(0, (n_c + (_NLANE - 1)) // _NLANE, _ties, 0)
            return 0

        lax.cond(n_c <= _NLANE, _fin_small, _fin_ties, (n_c, kk))

        pltpu.async_copy(orow, o_hbm.at[row], out_sem)

    pltpu.async_copy(x_hbm.at[base], xr0, in0)
    pltpu.async_copy(x_hbm.at[base + 1], xr1, in1)

    def pair_body(h, carry):
        r0 = base + 2 * h
        pltpu.make_async_copy(x_hbm.at[r0], xr0, in0).wait()
        _process(h, r0, xr0, or0, in0, out0)
        pltpu.make_async_copy(x_hbm.at[r0 + 1], xr1, in1).wait()
        _process(h, r0 + 1, xr1, or1, in1, out1)
        return carry

    lax.fori_loop(0, npairs, pair_body, 0)

    last = base + rows_per_worker - 2
    pltpu.make_async_copy(or0, o_hbm.at[last], out0).wait()
    pltpu.make_async_copy(or1, o_hbm.at[last + 1], out1).wait()


def kernel(x):
    B, N, C, L = x.shape
    k = int(L * _MASK_RATIO)
    M = B * N * C
    nw = 32
    rows_per_worker = M // nw
    nvec = L // _NLANE
    xf = x.reshape(M, L)

    sck = pl.kernel(
        functools.partial(_sc_body, k, rows_per_worker, nvec),
        out_type=jax.ShapeDtypeStruct((M, L), jnp.float32),
        mesh=plsc.VectorSubcoreMesh(core_axis_name="c", subcore_axis_name="s",
                                    num_cores=2, num_subcores=16),
        compiler_params=pltpu.CompilerParams(needs_layout_passes=False),
        scratch_types=[
            pltpu.VMEM((L,), jnp.float32),
            pltpu.VMEM((L,), jnp.float32),
            pltpu.VMEM((L,), jnp.float32),
            pltpu.VMEM((L,), jnp.float32),
            pltpu.VMEM((_NLANE * 273,), jnp.int32),
            pltpu.VMEM((256 + _NLANE,), jnp.int32),
            pltpu.VMEM((L + _NLANE,), jnp.int32),
            pltpu.VMEM((L + _NLANE,), jnp.int32),
            pltpu.VMEM((L + _NLANE,), jnp.int32),
            pltpu.VMEM((L + _NLANE,), jnp.int32),
            pltpu.VMEM((_NLANE * 19 + _NLANE,), jnp.int32),
            pltpu.VMEM((2 * _NLANE,), jnp.int32),
            pltpu.SemaphoreType.DMA,
            pltpu.SemaphoreType.DMA,
            pltpu.SemaphoreType.DMA,
            pltpu.SemaphoreType.DMA,
        ],
    )
    out = sck(xf)
    return out.reshape(B, N, C, L)


# speedup vs baseline: 3.2792x; 1.0220x over previous
import functools

import jax
import jax.numpy as jnp
from jax import lax
from jax.experimental import pallas as pl
from jax.experimental.pallas import tpu as pltpu
from jax.experimental.pallas import tpu_sc as plsc

_MASK_RATIO = 0.25
_NLANE = 16
_ROUNDS = ((19, 4), (15, 4), (11, 4), (7, 4), (3, 4), (0, 3))


def _tree_sum(vs):
    while len(vs) > 1:
        vs = [vs[i] + vs[i + 1] for i in range(0, len(vs) - 1, 2)] + (
            [vs[-1]] if len(vs) % 2 else [])
    return vs[0]


def _suffix(v):
    return lax.rev(jnp.cumsum(lax.rev(v, (0,))), (0,))


def _sc_body(k, rows_per_worker, nvec, x_hbm, o_hbm, xr0, xr1, or0, or1,
             h8, s8, ca, pa, cb, pb, h4, s4, in0, in1, out0, out1):
    c = lax.axis_index("c")
    s = lax.axis_index("s")
    wid = s * 2 + c
    lane = lax.iota(jnp.int32, _NLANE)
    lane273 = lane * 273
    lane19 = lane * 19
    lane16 = lane * _NLANE
    ones_i = jnp.ones((_NLANE,), jnp.int32)
    zeros_i = jnp.zeros((_NLANE,), jnp.int32)
    ones_f = jnp.ones((_NLANE,), jnp.float32)

    def _zero_h8(i, carry):
        h8[pl.ds(i * _NLANE, _NLANE)] = zeros_i
        return carry

    lax.fori_loop(0, (_NLANE * 273 + _NLANE - 1) // _NLANE, _zero_h8, 0)
    s8[pl.ds(256, _NLANE)] = zeros_i
    s4[pl.ds(_NLANE, _NLANE)] = zeros_i

    base = wid * rows_per_worker
    npairs = rows_per_worker // 2

    def _process(h, row, xrow, orow, in_sem, out_sem):
        def _abits(i):
            v = xrow[pl.ds(i * _NLANE, _NLANE)]
            return lax.bitcast_convert_type(v, jnp.int32) & 0x7FFFFFFF

        @plsc.parallel_loop(0, nvec, step=1, unroll=8)
        def _hist8(i):
            a = _abits(i)
            plsc.addupdate_scatter(h8, [lane273 + (a >> 23)], ones_i)

        for ch in range(16):
            vs = [h8[pl.ds(l * 273 + ch * _NLANE, _NLANE)]
                  for l in range(_NLANE)]
            for l in range(_NLANE):
                h8[pl.ds(l * 273 + ch * _NLANE, _NLANE)] = zeros_i
            s8[pl.ds(ch * _NLANE, _NLANE)] = _suffix(_tree_sum(vs))
        tot = plsc.load_gather(s8, [lane16])
        carry_v = _suffix(tot) - tot
        cnt_parts = []
        for ch in range(16):
            sufc = s8[pl.ds(ch * _NLANE, _NLANE)] + carry_v[ch]
            s8[pl.ds(ch * _NLANE, _NLANE)] = sufc
            cnt_parts.append((sufc >= k).astype(jnp.int32))
        cntge = jnp.sum(_tree_sum(cnt_parts))
        bkt = cntge - 1
        kk = jnp.int32(k) - s8[pl.ds(bkt + 1, _NLANE)][0]

        @pl.when(h > 0)
        def _drain():
            pltpu.make_async_copy(orow, o_hbm.at[row], out_sem).wait()

        @plsc.parallel_loop(0, nvec, step=1, unroll=4, carry=zeros_i)
        def _compress(i, ptrv):
            a = _abits(i)
            e = a >> 23
            orow[pl.ds(i * _NLANE, _NLANE)] = jnp.where(e > bkt, 1.0, 0.0)
            m = e == bkt
            idx = ptrv + jnp.cumsum(m.astype(jnp.int32)) - 1
            plsc.store_scatter(ca, [idx], a, mask=m)
            plsc.store_scatter(pa, [idx], i * _NLANE + lane, mask=m)
            return ptrv + plsc.all_reduce_population_count(m)

        n_c = _compress[0]

        @pl.when(h + 1 < npairs)
        def _prefetch():
            pltpu.async_copy(x_hbm.at[row + 2], xrow, in_sem)

        shift0 = _ROUNDS[0][0]
        for l in range(_NLANE):
            h4[pl.ds(l * 19, _NLANE)] = zeros_i
        nv0 = (n_c + _NLANE - 1) // _NLANE

        @plsc.parallel_loop(0, nv0, step=1, unroll=2)
        def _hist4(j, n_c=n_c):
            av = ca[pl.ds(j * _NLANE, _NLANE)]
            valid = (j * _NLANE + lane) < n_c
            nib = (av >> shift0) & 0xF
            plsc.addupdate_scatter(h4, [lane19 + nib], ones_i, mask=valid)

        acc = _tree_sum([h4[pl.ds(l * 19, _NLANE)] for l in range(_NLANE)])
        suf = _suffix(acc)
        s4[pl.ds(0, _NLANE)] = suf
        nstar = jnp.sum((suf >= kk).astype(jnp.int32)) - 1
        kk = kk - s4[pl.ds(nstar + 1, _NLANE)][0]

        @plsc.parallel_loop(0, nv0, step=1, unroll=2, carry=zeros_i)
        def _recomp0(j, ptrv, n_c=n_c, nstar=nstar):
            av = ca[pl.ds(j * _NLANE, _NLANE)]
            pv = pa[pl.ds(j * _NLANE, _NLANE)]
            valid = (j * _NLANE + lane) < n_c
            nib = (av >> shift0) & 0xF
            win = valid & (nib > nstar)
            plsc.store_scatter(orow, [pv], ones_f, mask=win)
            keep = valid & (nib == nstar)
            idx = ptrv + jnp.cumsum(keep.astype(jnp.int32)) - 1
            plsc.store_scatter(cb, [idx], av, mask=keep)
            plsc.store_scatter(pb, [idx], pv, mask=keep)
            return ptrv + plsc.all_reduce_population_count(keep)

        n_c = _recomp0[0]

        shift1 = _ROUNDS[1][0]
        for l in range(_NLANE):
            h4[pl.ds(l * 19, _NLANE)] = zeros_i
        nv1 = (n_c + _NLANE - 1) // _NLANE

        @plsc.parallel_loop(0, nv1, step=1, unroll=2)
        def _hist41(j, n_c=n_c):
            av = cb[pl.ds(j * _NLANE, _NLANE)]
            valid = (j * _NLANE + lane) < n_c
            nib = (av >> shift1) & 0xF
            plsc.addupdate_scatter(h4, [lane19 + nib], ones_i, mask=valid)

        acc = _tree_sum([h4[pl.ds(l * 19, _NLANE)] for l in range(_NLANE)])
        suf = _suffix(acc)
        s4[pl.ds(0, _NLANE)] = suf
        nstar = jnp.sum((suf >= kk).astype(jnp.int32)) - 1
        kk = kk - s4[pl.ds(nstar + 1, _NLANE)][0]

        @plsc.parallel_loop(0, nv1, step=1, unroll=2, carry=zeros_i)
        def _recomp1(j, ptrv, n_c=n_c, nstar=nstar):
            av = cb[pl.ds(j * _NLANE, _NLANE)]
            pv = pb[pl.ds(j * _NLANE, _NLANE)]
            valid = (j * _NLANE + lane) < n_c
            nib = (av >> shift1) & 0xF
            win = valid & (nib > nstar)
            plsc.store_scatter(orow, [pv], ones_f, mask=win)
            keep = valid & (nib == nstar)
            idx = ptrv + jnp.cumsum(keep.astype(jnp.int32)) - 1
            plsc.store_scatter(ca, [idx], av, mask=keep)
            plsc.store_scatter(pa, [idx], pv, mask=keep)
            return ptrv + plsc.all_reduce_population_count(keep)

        n_c = _recomp1[0]

        def _round_inplace(args, shift, width):
            n_c, kk = args
            wmask = (1 << width) - 1
            for l in range(_NLANE):
                h4[pl.ds(l * 19, _NLANE)] = zeros_i
            nv = (n_c + _NLANE - 1) // _NLANE

            def _h(j, carry):
                av = ca[pl.ds(j * _NLANE, _NLANE)]
                valid = (j * _NLANE + lane) < n_c
                nib = (av >> shift) & wmask
                plsc.addupdate_scatter(h4, [lane19 + nib], ones_i,
                                       mask=valid)
                return carry

            lax.fori_loop(0, nv, _h, 0)
            acc = _tree_sum([h4[pl.ds(l * 19, _NLANE)]
                             for l in range(_NLANE)])
            suf = _suffix(acc)
            s4[pl.ds(0, _NLANE)] = suf
            nstar = jnp.sum((suf >= kk).astype(jnp.int32)) - 1
            kk2 = kk - s4[pl.ds(nstar + 1, _NLANE)][0]

            def _rc(j, ptr):
                av = ca[pl.ds(j * _NLANE, _NLANE)]
                pv = pa[pl.ds(j * _NLANE, _NLANE)]
                valid = (j * _NLANE + lane) < n_c
                nib = (av >> shift) & wmask
                win = valid & (nib > nstar)
                plsc.store_scatter(orow, [pv], ones_f, mask=win)
                keep = valid & (nib == nstar)
                idx = ptr + jnp.cumsum(keep.astype(jnp.int32)) - 1
                plsc.store_scatter(ca, [idx], av, mask=keep)
                plsc.store_scatter(pa, [idx], pv, mask=keep)
                return ptr + jnp.sum(keep.astype(jnp.int32))

            n_c2 = lax.fori_loop(0, nv, _rc, jnp.int32(0))
            return n_c2, kk2

        state = (n_c, kk)
        for shift, width in _ROUNDS[2:]:
            state = lax.cond(
                state[0] > _NLANE,
                functools.partial(_round_inplace, shift=shift, width=width),
                lambda args: args,
                state)
        n_c, kk = state

        def _fin_small(args):
            n_c, kk = args
            av = ca[pl.ds(0, _NLANE)]
            pv = pa[pl.ds(0, _NLANE)]
            valid = lane < n_c
            srt = jnp.sort(jnp.where(valid, av, 0))
            s4[pl.ds(0, _NLANE)] = srt
            thr = s4[pl.ds(_NLANE - kk, _NLANE)][0]
            win = valid & (av >= thr)
            plsc.store_scatter(orow, [pv], ones_f, mask=win)
            return 0

        def _fin_ties(args):
            n_c, _ = args

            def _ties(j, carry):
                pv = pa[pl.ds(j * _NLANE, _NLANE)]
                valid = (j * _NLANE + lane) < n_c
                plsc.store_scatter(orow, [pv], ones_f, mask=valid)
                return carry

            lax.fori_loop(0, (n_c + (_NLANE - 1)) // _NLANE, _ties, 0)
            return 0

        lax.cond(n_c <= _NLANE, _fin_small, _fin_ties, (n_c, kk))

        pltpu.async_copy(orow, o_hbm.at[row], out_sem)

    pltpu.async_copy(x_hbm.at[base], xr0, in0)
    pltpu.async_copy(x_hbm.at[base + 1], xr1, in1)

    def pair_body(h, carry):
        r0 = base + 2 * h
        pltpu.make_async_copy(x_hbm.at[r0], xr0, in0).wait()
        _process(h, r0, xr0, or0, in0, out0)
        pltpu.make_async_copy(x_hbm.at[r0 + 1], xr1, in1).wait()
        _process(h, r0 + 1, xr1, or1, in1, out1)
        return carry

    lax.fori_loop(0, npairs, pair_body, 0)

    last = base + rows_per_worker - 2
    pltpu.make_async_copy(or0, o_hbm.at[last], out0).wait()
    pltpu.make_async_copy(or1, o_hbm.at[last + 1], out1).wait()


def kernel(x):
    B, N, C, L = x.shape
    k = int(L * _MASK_RATIO)
    M = B * N * C
    nw = 32
    rows_per_worker = M // nw
    nvec = L // _NLANE
    xf = x.reshape(M, L)

    sck = pl.kernel(
        functools.partial(_sc_body, k, rows_per_worker, nvec),
        out_type=jax.ShapeDtypeStruct((M, L), jnp.float32),
        mesh=plsc.VectorSubcoreMesh(core_axis_name="c", subcore_axis_name="s",
                                    num_cores=2, num_subcores=16),
        compiler_params=pltpu.CompilerParams(needs_layout_passes=False),
        scratch_types=[
            pltpu.VMEM((L,), jnp.float32),
            pltpu.VMEM((L,), jnp.float32),
            pltpu.VMEM((L,), jnp.float32),
            pltpu.VMEM((L,), jnp.float32),
            pltpu.VMEM((_NLANE * 273,), jnp.int32),
            pltpu.VMEM((256 + _NLANE,), jnp.int32),
            pltpu.VMEM((L + _NLANE,), jnp.int32),
            pltpu.VMEM((L + _NLANE,), jnp.int32),
            pltpu.VMEM((L + _NLANE,), jnp.int32),
            pltpu.VMEM((L + _NLANE,), jnp.int32),
            pltpu.VMEM((_NLANE * 19 + _NLANE,), jnp.int32),
            pltpu.VMEM((2 * _NLANE,), jnp.int32),
            pltpu.SemaphoreType.DMA,
            pltpu.SemaphoreType.DMA,
            pltpu.SemaphoreType.DMA,
            pltpu.SemaphoreType.DMA,
        ],
    )
    out = sck(xf)
    return out.reshape(B, N, C, L)
